# Initial kernel scaffold; baseline (speedup 1.0000x reference)
import jax
import jax.numpy as jnp
from jax.experimental import pallas as pl


def kernel(x_nchw, w_up, b_up, conv1_w, conv1_scale, conv1_shift, conv2_w, conv2_scale, conv2_shift):
    raise NotImplementedError("write your pallas kernel here")


# speedup vs baseline: 1.4692x; 1.4692x over previous
import jax
import jax.numpy as jnp
from jax.experimental import pallas as pl
from jax.experimental.pallas import tpu as pltpu


def _upsample_kernel(x_ref, w_ref, b_ref, o_ref):
    cin = x_ref.shape[1]
    h = o_ref.shape[1]
    w_dim = o_ref.shape[3]
    two_cout = o_ref.shape[4]
    xc = x_ref[0].astype(jnp.bfloat16)
    y = jax.lax.dot_general(
        xc, w_ref[...], (((0,), (0,)), ((), ())),
        preferred_element_type=jnp.float32,
    ) + b_ref[...]
    yb = y.astype(jnp.bfloat16)
    for a in range(2):
        ya = yb[:, a * two_cout:(a + 1) * two_cout]
        o_ref[0, :, a, :, :] = ya.reshape(h, w_dim, two_cout)


def _double_conv_kernel(x_ref, w1_ref, s1_ref, t1_ref, w2_ref, s2_ref, t2_ref,
                        o_ref, xpad_ref):
    _, ho, wo, c = x_ref.shape
    cout = o_ref.shape[1]

    xpad_ref[0:1, :, :] = jnp.zeros((1, wo + 2, c), jnp.bfloat16)
    xpad_ref[ho + 1:ho + 2, :, :] = jnp.zeros((1, wo + 2, c), jnp.bfloat16)
    xpad_ref[:, 0:1, :] = jnp.zeros((ho + 2, 1, c), jnp.bfloat16)
    xpad_ref[:, wo + 1:wo + 2, :] = jnp.zeros((ho + 2, 1, c), jnp.bfloat16)
    xpad_ref[1:ho + 1, 1:wo + 1, :] = x_ref[0]

    def conv(w_ref_, s_ref_, t_ref_):
        acc = jnp.zeros((ho * wo, cout), jnp.float32)
        for dh in range(3):
            a_dh = jnp.concatenate(
                [xpad_ref[dh:dh + ho, dw:dw + wo, :] for dw in range(3)],
                axis=-1,
            ).reshape(ho * wo, 3 * c)
            acc = acc + jnp.dot(a_dh, w_ref_[dh],
                                preferred_element_type=jnp.float32)
        return jnp.maximum(acc * s_ref_[...] + t_ref_[...], 0.0)

    y1 = conv(w1_ref, s1_ref, t1_ref)
    xpad_ref[1:ho + 1, 1:wo + 1, :] = (
        y1.astype(jnp.bfloat16).reshape(ho, wo, cout))
    y2 = conv(w2_ref, s2_ref, t2_ref)
    o_ref[0] = y2.T


def kernel(x_nchw, w_up, b_up, conv1_w, conv1_scale, conv1_shift,
           conv2_w, conv2_scale, conv2_shift):
    n, cin, h, w = x_nchw.shape
    cout = w_up.shape[1]
    ho, wo = 2 * h, 2 * w

    w2d = jnp.transpose(w_up, (0, 2, 3, 1)).reshape(cin, 4 * cout)
    w2d = w2d.astype(jnp.bfloat16)
    b2d = jnp.tile(b_up, 4).reshape(1, 4 * cout)
    w1 = conv1_w.reshape(3, 3 * cout, cout).astype(jnp.bfloat16)
    w2 = conv2_w.reshape(3, 3 * cout, cout).astype(jnp.bfloat16)
    s1 = conv1_scale.reshape(1, cout)
    t1 = conv1_shift.reshape(1, cout)
    s2 = conv2_scale.reshape(1, cout)
    t2 = conv2_shift.reshape(1, cout)

    x_flat = x_nchw.reshape(n, cin, h * w)

    up = pl.pallas_call(
        _upsample_kernel,
        out_shape=jax.ShapeDtypeStruct((n, h, 2, w, 2 * cout), jnp.bfloat16),
        grid=(n,),
        in_specs=[
            pl.BlockSpec((1, cin, h * w), lambda i: (i, 0, 0)),
            pl.BlockSpec((cin, 4 * cout), lambda i: (0, 0)),
            pl.BlockSpec((1, 4 * cout), lambda i: (0, 0)),
        ],
        out_specs=pl.BlockSpec((1, h, 2, w, 2 * cout),
                               lambda i: (i, 0, 0, 0, 0)),
        compiler_params=pltpu.CompilerParams(
            dimension_semantics=("parallel",)),
    )(x_flat, w2d, b2d)
    fine = up.reshape(n, ho, wo, cout)

    out = pl.pallas_call(
        _double_conv_kernel,
        out_shape=jax.ShapeDtypeStruct((n, cout, ho * wo), jnp.float32),
        grid=(n,),
        in_specs=[
            pl.BlockSpec((1, ho, wo, cout), lambda i: (i, 0, 0, 0)),
            pl.BlockSpec((3, 3 * cout, cout), lambda i: (0, 0, 0)),
            pl.BlockSpec((1, cout), lambda i: (0, 0)),
            pl.BlockSpec((1, cout), lambda i: (0, 0)),
            pl.BlockSpec((3, 3 * cout, cout), lambda i: (0, 0, 0)),
            pl.BlockSpec((1, cout), lambda i: (0, 0)),
            pl.BlockSpec((1, cout), lambda i: (0, 0)),
        ],
        out_specs=pl.BlockSpec((1, cout, ho * wo), lambda i: (i, 0, 0)),
        scratch_shapes=[pltpu.VMEM((ho + 2, wo + 2, cout), jnp.bfloat16)],
        compiler_params=pltpu.CompilerParams(
            dimension_semantics=("parallel",)),
    )(fine, w1, s1, t1, w2, s2, t2)

    return out.reshape(n, cout, ho, wo)


# speedup vs baseline: 1.4987x; 1.0201x over previous
import functools

import jax
import jax.numpy as jnp
from jax.experimental import pallas as pl
from jax.experimental.pallas import tpu as pltpu


def _upsample_kernel(x_ref, w_ref, b_ref, o_ref):
    cin = x_ref.shape[1]
    h = o_ref.shape[1]
    w_dim = o_ref.shape[3]
    two_cout = o_ref.shape[4]
    xc = x_ref[0].astype(jnp.bfloat16)
    y = jax.lax.dot_general(
        xc, w_ref[...], (((0,), (0,)), ((), ())),
        preferred_element_type=jnp.float32,
    ) + b_ref[...]
    yb = y.astype(jnp.bfloat16)
    for a in range(2):
        ya = yb[:, a * two_cout:(a + 1) * two_cout]
        o_ref[0, :, a, :, :] = ya.reshape(h, w_dim, two_cout)


def _double_conv_kernel(x_ref, w1_ref, s1_ref, t1_ref, w2_ref, s2_ref, t2_ref,
                        o_ref, xs_ref, *, wo):
    _, hw, c = x_ref.shape
    cout = o_ref.shape[1]

    x_cm = x_ref[0].T

    col = jax.lax.broadcasted_iota(jnp.int32, (1, hw), 1) % wo
    left_edge = col == 0
    right_edge = col == wo - 1

    def shift(xs, s):
        if s > 0:
            return jnp.concatenate(
                [xs[:, s:], jnp.zeros((xs.shape[0], s), xs.dtype)], axis=1)
        if s < 0:
            return jnp.concatenate(
                [jnp.zeros((xs.shape[0], -s), xs.dtype), xs[:, :s]], axis=1)
        return xs

    def conv(xcm, w_ref_, s_, t_):
        zero = jnp.zeros_like(xcm)
        pick = {-1: jnp.where(right_edge, zero, xcm),
                0: xcm,
                1: jnp.where(left_edge, zero, xcm)}
        acc = jnp.zeros((cout, hw), jnp.float32)
        k = 0
        for dh in (-1, 0, 1):
            for dw in (-1, 0, 1):
                xs_ref[k % 2] = shift(pick[dw], dh * wo + dw)
                acc = acc + jnp.dot(w_ref_[k], xs_ref[k % 2],
                                    preferred_element_type=jnp.float32)
                k += 1
        return jnp.maximum(acc * s_ + t_, 0.0)

    y1 = conv(x_cm, w1_ref, s1_ref[...], t1_ref[...])
    y2 = conv(y1.astype(jnp.bfloat16), w2_ref, s2_ref[...], t2_ref[...])
    o_ref[0] = y2


def kernel(x_nchw, w_up, b_up, conv1_w, conv1_scale, conv1_shift,
           conv2_w, conv2_scale, conv2_shift):
    n, cin, h, w = x_nchw.shape
    cout = w_up.shape[1]
    ho, wo = 2 * h, 2 * w

    w2d = jnp.transpose(w_up, (0, 2, 3, 1)).reshape(cin, 4 * cout)
    w2d = w2d.astype(jnp.bfloat16)
    b2d = jnp.tile(b_up, 4).reshape(1, 4 * cout)
    w1 = jnp.transpose(conv1_w, (0, 1, 3, 2)).reshape(9, cout, cout)
    w1 = w1.astype(jnp.bfloat16)
    w2 = jnp.transpose(conv2_w, (0, 1, 3, 2)).reshape(9, cout, cout)
    w2 = w2.astype(jnp.bfloat16)
    s1 = conv1_scale.reshape(cout, 1)
    t1 = conv1_shift.reshape(cout, 1)
    s2 = conv2_scale.reshape(cout, 1)
    t2 = conv2_shift.reshape(cout, 1)

    x_flat = x_nchw.reshape(n, cin, h * w)

    up = pl.pallas_call(
        _upsample_kernel,
        out_shape=jax.ShapeDtypeStruct((n, h, 2, w, 2 * cout), jnp.bfloat16),
        grid=(n,),
        in_specs=[
            pl.BlockSpec((1, cin, h * w), lambda i: (i, 0, 0)),
            pl.BlockSpec((cin, 4 * cout), lambda i: (0, 0)),
            pl.BlockSpec((1, 4 * cout), lambda i: (0, 0)),
        ],
        out_specs=pl.BlockSpec((1, h, 2, w, 2 * cout),
                               lambda i: (i, 0, 0, 0, 0)),
        compiler_params=pltpu.CompilerParams(
            dimension_semantics=("parallel",)),
    )(x_flat, w2d, b2d)
    fine = up.reshape(n, ho * wo, cout)

    out = pl.pallas_call(
        functools.partial(_double_conv_kernel, wo=wo),
        out_shape=jax.ShapeDtypeStruct((n, cout, ho * wo), jnp.float32),
        grid=(n,),
        in_specs=[
            pl.BlockSpec((1, ho * wo, cout), lambda i: (i, 0, 0)),
            pl.BlockSpec((9, cout, cout), lambda i: (0, 0, 0)),
            pl.BlockSpec((cout, 1), lambda i: (0, 0)),
            pl.BlockSpec((cout, 1), lambda i: (0, 0)),
            pl.BlockSpec((9, cout, cout), lambda i: (0, 0, 0)),
            pl.BlockSpec((cout, 1), lambda i: (0, 0)),
            pl.BlockSpec((cout, 1), lambda i: (0, 0)),
        ],
        out_specs=pl.BlockSpec((1, cout, ho * wo), lambda i: (i, 0, 0)),
        scratch_shapes=[pltpu.VMEM((2, cout, ho * wo), jnp.bfloat16)],
        compiler_params=pltpu.CompilerParams(
            dimension_semantics=("parallel",)),
    )(fine, w1, s1, t1, w2, s2, t2)

    return out.reshape(n, cout, ho, wo)
